# baseline (speedup 1.0000x reference)
import functools

import jax
import jax.numpy as jnp
from jax import lax
from jax.experimental import pallas as pl
from jax.experimental.pallas import tpu as pltpu
from jax.experimental.pallas import tpu_sc as plsc

B = 1024
L = 50
K = 32
PHI = 3
VOCAB = 1000000
VOCAB_PHI = VOCAB * PHI
N = B * L
NC = 2
NS = 16
NW = NC * NS
BPW = N // NW
CHUNK = 80
NCH = BPW // CHUNK


def _sc_gather_body(idx_hbm, idx3_hbm, mu_hbm, ls_hbm, phif_hbm,
                    mu_out, ls_out, phi_out,
                    idx_v, idx3_v, mu_v, ls_v, phi_v, sem):
  wid = lax.axis_index("s") * NC + lax.axis_index("c")
  base = wid * BPW
  pltpu.sync_copy(idx_hbm.at[wid], idx_v)
  pltpu.sync_copy(idx3_hbm.at[:, wid], idx3_v)
  copies = []
  for c in range(NCH):
    row = pl.ds(c * CHUNK, CHUNK)
    copies.append(pltpu.async_copy(mu_hbm.at[idx_v.at[c]], mu_v.at[row], sem))
    copies.append(pltpu.async_copy(ls_hbm.at[idx_v.at[c]], ls_v.at[row], sem))
    for k in range(PHI):
      copies.append(pltpu.async_copy(phif_hbm.at[idx3_v.at[k, c]],
                                     phi_v.at[k, row], sem))
  for cp in copies:
    cp.wait()
  out_rows = pl.ds(base, BPW)
  pltpu.sync_copy(mu_v, mu_out.at[out_rows])
  pltpu.sync_copy(ls_v, ls_out.at[out_rows])
  pltpu.sync_copy(phi_v, phi_out.at[:, out_rows])


def _make_sc_gather():
  mesh = plsc.VectorSubcoreMesh(core_axis_name="c", subcore_axis_name="s")
  return pl.kernel(
      _sc_gather_body,
      mesh=mesh,
      out_type=[
          jax.ShapeDtypeStruct((N, K), jnp.float32),
          jax.ShapeDtypeStruct((N, K), jnp.float32),
          jax.ShapeDtypeStruct((PHI, N), jnp.float32),
      ],
      scratch_types=[
          pltpu.VMEM((NCH, CHUNK), jnp.int32),
          pltpu.VMEM((PHI, NCH, CHUNK), jnp.int32),
          pltpu.VMEM((BPW, K), jnp.float32),
          pltpu.VMEM((BPW, K), jnp.float32),
          pltpu.VMEM((PHI, BPW), jnp.float32),
          pltpu.SemaphoreType.DMA,
      ],
      compiler_params=pltpu.CompilerParams(use_tc_tiling_on_sc=False),
  )


ROWS = 256


def _expand_body(ls_ref, e_ref, out_ref):
  sd = jnp.exp(ls_ref[...])
  out_ref[...] = lax.dot_general(
      sd, e_ref[...], (((1,), (0,)), ((), ())),
      precision=lax.Precision.HIGHEST)


def _expand(ls_flat, e_mat):
  return pl.pallas_call(
      _expand_body,
      grid=(N // ROWS,),
      in_specs=[
          pl.BlockSpec((ROWS, K), lambda i: (i, 0)),
          pl.BlockSpec((K, K * K), lambda i: (0, 0)),
      ],
      out_specs=pl.BlockSpec((ROWS, K * K), lambda i: (i, 0)),
      out_shape=jax.ShapeDtypeStruct((N, K * K), jnp.float32),
  )(ls_flat, e_mat)


def kernel(token_ids, mu_table, log_sigma_diag, phi_table):
  tok = token_ids.reshape(N)
  idx = tok.reshape(NW, NCH, CHUNK)
  idx3 = (tok[None, :] * PHI + jnp.arange(PHI, dtype=token_ids.dtype)[:, None]
          ).reshape(PHI, NW, NCH, CHUNK)
  mu_flat, ls_flat, phi_t = _make_sc_gather()(
      idx, idx3, mu_table, log_sigma_diag, phi_table.reshape(VOCAB_PHI))
  cols = lax.broadcasted_iota(jnp.int32, (K, K * K), 1)
  rows = lax.broadcasted_iota(jnp.int32, (K, K * K), 0)
  e_mat = (cols == rows * (K + 1)).astype(jnp.float32)
  sigma2d = _expand(ls_flat, e_mat)
  return (mu_flat.reshape(B, L, K),
          sigma2d.reshape(B, L, K, K),
          phi_t.T.reshape(B, L, PHI))
